# baseline (speedup 1.0000x reference)
import functools

import jax
import jax.numpy as jnp
from jax import lax
from jax.experimental import pallas as pl
from jax.experimental.pallas import tpu as pltpu
from jax.experimental.pallas import tpu_sc as plsc

B, N, M, K = 8, 8192, 256, 32
import numpy as np

_BIG = np.float32(1e10)
_INF = np.float32(3.0e38)


def _fps_kernel(x0_ref, x1_ref, x2_ref, c0_ref, c1_ref, c2_ref, dist_ref):
    x0 = x0_ref[...]
    x1 = x1_ref[...]
    x2 = x2_ref[...]
    colN = lax.broadcasted_iota(jnp.int32, (B, N), 1)
    colM = lax.broadcasted_iota(jnp.int32, (B, M), 1)
    dist_ref[...] = jnp.full((B, N), _BIG, jnp.float32)

    def body(i, far):
        m = colN == far
        c0 = jnp.sum(jnp.where(m, x0, 0.0), axis=1, keepdims=True)
        c1 = jnp.sum(jnp.where(m, x1, 0.0), axis=1, keepdims=True)
        c2 = jnp.sum(jnp.where(m, x2, 0.0), axis=1, keepdims=True)
        sel = colM == i
        c0_ref[...] = jnp.where(sel, c0, c0_ref[...])
        c1_ref[...] = jnp.where(sel, c1, c1_ref[...])
        c2_ref[...] = jnp.where(sel, c2, c2_ref[...])
        d = (x0 - c0) ** 2 + (x1 - c1) ** 2 + (x2 - c2) ** 2
        dist = jnp.minimum(dist_ref[...], d)
        dist_ref[...] = dist
        maxv = jnp.max(dist, axis=1, keepdims=True)
        far_new = jnp.min(
            jnp.where(dist == maxv, colN, N), axis=1, keepdims=True
        )
        return far_new.astype(jnp.int32)

    far0 = jnp.zeros((B, 1), jnp.int32)
    lax.fori_loop(0, M, body, far0)


def _fps(x0, x1, x2):
    return pl.pallas_call(
        _fps_kernel,
        out_shape=[jax.ShapeDtypeStruct((B, M), jnp.float32)] * 3,
        scratch_shapes=[pltpu.VMEM((B, N), jnp.float32)],
    )(x0, x1, x2)


NCH = 64
CW = N // NCH
CAND = 128


def _dist_kernel(x0_ref, x1_ref, x2_ref, c0_ref, c1_ref, c2_ref,
                 dout_ref, t_ref):
    x0 = x0_ref[0]
    x1 = x1_ref[0]
    x2 = x2_ref[0]
    c0 = c0_ref[0]
    c1 = c1_ref[0]
    c2 = c2_ref[0]
    d = (c0 - x0) ** 2 + (c1 - x1) ** 2 + (c2 - x2) ** 2
    dout_ref[0] = d
    w = jnp.min(d.reshape(M, NCH, CW), axis=2)
    colC = lax.broadcasted_iota(jnp.int32, (M, NCH), 1)

    def body(s, carry):
        w, _ = carry
        minv = jnp.min(w, axis=1, keepdims=True)
        sel = jnp.min(jnp.where(w == minv, colC, NCH), axis=1, keepdims=True)
        w = jnp.where(colC == sel, _INF, w)
        return (w, minv)

    _, t = lax.fori_loop(0, K, body, (w, jnp.zeros((M, 1), jnp.float32)))
    t_ref[0] = jnp.broadcast_to(t, (M, 16))


def _dist(x0, x1, x2, c0t, c1t, c2t):
    nb = x0.shape[0]
    return pl.pallas_call(
        _dist_kernel,
        grid=(nb,),
        in_specs=[
            pl.BlockSpec((1, 1, N), lambda b: (b, 0, 0)),
            pl.BlockSpec((1, 1, N), lambda b: (b, 0, 0)),
            pl.BlockSpec((1, 1, N), lambda b: (b, 0, 0)),
            pl.BlockSpec((1, M, 1), lambda b: (b, 0, 0)),
            pl.BlockSpec((1, M, 1), lambda b: (b, 0, 0)),
            pl.BlockSpec((1, M, 1), lambda b: (b, 0, 0)),
        ],
        out_specs=[
            pl.BlockSpec((1, M, N), lambda b: (b, 0, 0)),
            pl.BlockSpec((1, M, 16), lambda b: (b, 0, 0)),
        ],
        out_shape=[
            jax.ShapeDtypeStruct((nb, M, N), jnp.float32),
            jax.ShapeDtypeStruct((nb, M, 16), jnp.float32),
        ],
    )(x0, x1, x2, c0t, c1t, c2t)


_NC, _NS = 2, 16
_NW = _NC * _NS
_RPW = B * M // _NW


def _sc_filter_kernel(rpw, d_hbm, t_hbm, cd_hbm, ci_hbm,
                      d_v, t_v, cd_v, ci_v, semA, semB):
    wid = lax.axis_index("s") * _NC + lax.axis_index("c")
    r0 = wid * rpw
    inf16 = jnp.full((16,), _INF, jnp.float32)
    iota16 = lax.iota(jnp.int32, 16)

    pltpu.sync_copy(t_hbm.at[pl.ds(r0 * 16, rpw * 16)], t_v)

    def init_body(i, _):
        cd_v[pl.ds(i * 16, 16)] = inf16
        return 0

    lax.fori_loop(0, rpw * CAND // 16, init_body, 0)

    def filter_row(r, buf):
        tv = t_v[pl.ds(r * 16, 16)]
        obase = r * CAND
        nq = 8

        def chunk(j, cnt):
            base = j * (16 * nq)
            dvs = [d_v[buf, pl.ds(base + q * 16, 16)] for q in range(nq)]
            ms = [dv <= tv for dv in dvs]
            mors = ms[0]
            for q in range(1, nq):
                mors = mors | ms[q]
            npass = plsc.all_reduce_population_count(mors)
            anyv = npass[0] != 0

            def heavy(c):
                for q in range(nq):
                    mi = ms[q].astype(jnp.int32)
                    cs = plsc.cumsum(mi)
                    pos = c + cs - 1
                    ok = ms[q] & (pos < CAND)
                    plsc.store_scatter(cd_v, [obase + pos], dvs[q], mask=ok)
                    plsc.store_scatter(
                        ci_v, [obase + pos], base + q * 16 + iota16, mask=ok)
                    c = c + jnp.sum(mi)
                return c

            return lax.cond(anyv, heavy, lambda c: c, cnt)

        lax.fori_loop(0, N // (16 * nq), chunk, 0)
        return 0

    pltpu.async_copy(d_hbm.at[r0], d_v.at[0], semA)

    def pair(p, _):
        ra = r0 + 2 * p
        pltpu.async_copy(d_hbm.at[ra + 1], d_v.at[1], semB)
        pltpu.make_async_copy(d_hbm.at[ra], d_v.at[0], semA).wait()
        filter_row(2 * p, 0)
        rn = jnp.minimum(ra + 2, r0 + rpw - 1)
        pltpu.async_copy(d_hbm.at[rn], d_v.at[0], semA)
        pltpu.make_async_copy(
            d_hbm.at[ra + 1], d_v.at[1], semB).wait()
        filter_row(2 * p + 1, 1)
        return 0

    lax.fori_loop(0, rpw // 2, pair, 0)
    pltpu.make_async_copy(d_hbm.at[r0], d_v.at[0], semA).wait()

    pltpu.sync_copy(cd_v, cd_hbm.at[pl.ds(r0 * CAND, rpw * CAND)])
    pltpu.sync_copy(ci_v, ci_hbm.at[pl.ds(r0 * CAND, rpw * CAND)])


def _sc_filter(dflat, tflat):
    nrows = dflat.shape[0]
    rpw = nrows // _NW
    mesh = plsc.VectorSubcoreMesh(core_axis_name="c", subcore_axis_name="s")
    f = functools.partial(
        pl.kernel,
        mesh=mesh,
        compiler_params=pltpu.CompilerParams(needs_layout_passes=False),
        out_type=[
            jax.ShapeDtypeStruct((nrows * CAND,), jnp.float32),
            jax.ShapeDtypeStruct((nrows * CAND,), jnp.int32),
        ],
        scratch_types=[
            pltpu.VMEM((2, N), jnp.float32),
            pltpu.VMEM((rpw * 16,), jnp.float32),
            pltpu.VMEM((rpw * CAND,), jnp.float32),
            pltpu.VMEM((rpw * CAND,), jnp.int32),
            pltpu.SemaphoreType.DMA,
            pltpu.SemaphoreType.DMA,
        ],
    )(functools.partial(_sc_filter_kernel, rpw))
    return f(dflat, tflat)


def _sel_kernel(cd_ref, ci_ref, idx_ref):
    ic = ci_ref[0]
    colK = lax.broadcasted_iota(jnp.int32, (M, K), 1)

    def body(s, dc):
        minv = jnp.min(dc, axis=1, keepdims=True)
        sel = jnp.min(jnp.where(dc == minv, ic, N), axis=1, keepdims=True)
        dc = jnp.where((dc == minv) & (ic == sel), _INF, dc)
        idx_ref[0] = jnp.where(colK == s, sel, idx_ref[0])
        return dc

    lax.fori_loop(0, K, body, cd_ref[0])


def _sel(cd, ci):
    return pl.pallas_call(
        _sel_kernel,
        grid=(B,),
        in_specs=[
            pl.BlockSpec((1, M, CAND), lambda b: (b, 0, 0)),
            pl.BlockSpec((1, M, CAND), lambda b: (b, 0, 0)),
        ],
        out_specs=pl.BlockSpec((1, M, K), lambda b: (b, 0, 0)),
        out_shape=jax.ShapeDtypeStruct((B, M, K), jnp.int32),
    )(cd, ci)


_NC, _NS = 2, 16
_NW = _NC * _NS
_ROWS = B * M * K // _NW
_TPB = _NW // B
_D = 16


def _sc_gather_kernel(xpad_hbm, idx_hbm, out_hbm, idx_v, rows_v, sem):
    wid = lax.axis_index("s") * _NC + lax.axis_index("c")
    b = wid // _TPB
    base = b * (M * K) + (wid % _TPB) * _ROWS
    pltpu.sync_copy(idx_hbm.at[pl.ds(base, _ROWS)], idx_v)
    bn = b * N

    def body(j, _):
        idx_v[pl.ds(j * 16, 16)] = idx_v[pl.ds(j * 16, 16)] + bn
        return 0

    lax.fori_loop(0, _ROWS // 16, body, 0)
    pltpu.async_copy(xpad_hbm.at[idx_v], rows_v, sem).wait()
    pltpu.sync_copy(rows_v, out_hbm.at[pl.ds(base, _ROWS)])


def _sc_gather(xpad, idxf):
    mesh = plsc.VectorSubcoreMesh(core_axis_name="c", subcore_axis_name="s")
    f = functools.partial(
        pl.kernel,
        mesh=mesh,
        compiler_params=pltpu.CompilerParams(use_tc_tiling_on_sc=False),
        out_type=jax.ShapeDtypeStruct((B * M * K, _D), jnp.float32),
        scratch_types=[
            pltpu.VMEM((_ROWS,), jnp.int32),
            pltpu.VMEM((_ROWS, _D), jnp.float32),
            pltpu.SemaphoreType.DMA,
        ],
    )(_sc_gather_kernel)
    return f(xpad, idxf)


def kernel(x):
    x0 = x[:, :, 0]
    x1 = x[:, :, 1]
    x2 = x[:, :, 2]
    c0, c1, c2 = _fps(x0, x1, x2)
    cds, cis = [], []
    for h in range(2):
        sl = slice(h * (B // 2), (h + 1) * (B // 2))
        dm, tm = _dist(
            x0[sl, None, :], x1[sl, None, :], x2[sl, None, :],
            c0[sl, :, None], c1[sl, :, None], c2[sl, :, None],
        )
        cdh, cih = _sc_filter(dm.reshape(B // 2 * M, N), tm.reshape(-1))
        cds.append(cdh)
        cis.append(cih)
    cd = jnp.concatenate(cds)
    ci = jnp.concatenate(cis)
    idx = _sel(
        cd.reshape(B, M, CAND), ci.reshape(B, M, CAND)
    )
    xpad = jnp.pad(x.reshape(B * N, 3), ((0, 0), (0, _D - 3)))
    out = _sc_gather(xpad, idx.reshape(-1))
    return out[:, :3].reshape(B, M, K, 3)


# speedup vs baseline: 1.0332x; 1.0332x over previous
import functools

import jax
import jax.numpy as jnp
from jax import lax
from jax.experimental import pallas as pl
from jax.experimental.pallas import tpu as pltpu
from jax.experimental.pallas import tpu_sc as plsc

B, N, M, K = 8, 8192, 256, 32
import numpy as np

_BIG = np.float32(1e10)
_INF = np.float32(3.0e38)


def _fps_kernel(x0_ref, x1_ref, x2_ref, c0_ref, c1_ref, c2_ref, dist_ref):
    x0 = x0_ref[...]
    x1 = x1_ref[...]
    x2 = x2_ref[...]
    colN = lax.broadcasted_iota(jnp.int32, (B, N), 1)
    colM = lax.broadcasted_iota(jnp.int32, (B, M), 1)
    dist_ref[...] = jnp.full((B, N), _BIG, jnp.float32)

    def body(i, far):
        m = colN == far
        c0 = jnp.sum(jnp.where(m, x0, 0.0), axis=1, keepdims=True)
        c1 = jnp.sum(jnp.where(m, x1, 0.0), axis=1, keepdims=True)
        c2 = jnp.sum(jnp.where(m, x2, 0.0), axis=1, keepdims=True)
        sel = colM == i
        c0_ref[...] = jnp.where(sel, c0, c0_ref[...])
        c1_ref[...] = jnp.where(sel, c1, c1_ref[...])
        c2_ref[...] = jnp.where(sel, c2, c2_ref[...])
        d = (x0 - c0) ** 2 + (x1 - c1) ** 2 + (x2 - c2) ** 2
        dist = jnp.minimum(dist_ref[...], d)
        dist_ref[...] = dist
        maxv = jnp.max(dist, axis=1, keepdims=True)
        far_new = jnp.min(
            jnp.where(dist == maxv, colN, N), axis=1, keepdims=True
        )
        return far_new.astype(jnp.int32)

    far0 = jnp.zeros((B, 1), jnp.int32)
    lax.fori_loop(0, M, body, far0)


def _fps(x0, x1, x2):
    return pl.pallas_call(
        _fps_kernel,
        out_shape=[jax.ShapeDtypeStruct((B, M), jnp.float32)] * 3,
        scratch_shapes=[pltpu.VMEM((B, N), jnp.float32)],
    )(x0, x1, x2)


NCH = 64
CW = N // NCH
CAND = 128


def _dist_kernel(x0_ref, x1_ref, x2_ref, c0_ref, c1_ref, c2_ref,
                 dout_ref, t_ref):
    x0 = x0_ref[0]
    x1 = x1_ref[0]
    x2 = x2_ref[0]
    c0 = c0_ref[0]
    c1 = c1_ref[0]
    c2 = c2_ref[0]
    d = (c0 - x0) ** 2 + (c1 - x1) ** 2 + (c2 - x2) ** 2
    dout_ref[0] = d
    w = jnp.min(d.reshape(M, NCH, CW), axis=2)
    colC = lax.broadcasted_iota(jnp.int32, (M, NCH), 1)

    def body(s, carry):
        w, _ = carry
        minv = jnp.min(w, axis=1, keepdims=True)
        sel = jnp.min(jnp.where(w == minv, colC, NCH), axis=1, keepdims=True)
        w = jnp.where(colC == sel, _INF, w)
        return (w, minv)

    _, t = lax.fori_loop(0, K, body, (w, jnp.zeros((M, 1), jnp.float32)))
    t_ref[0] = jnp.broadcast_to(t, (M, 16))


def _dist(x0, x1, x2, c0t, c1t, c2t):
    nb = x0.shape[0]
    return pl.pallas_call(
        _dist_kernel,
        grid=(nb,),
        in_specs=[
            pl.BlockSpec((1, 1, N), lambda b: (b, 0, 0)),
            pl.BlockSpec((1, 1, N), lambda b: (b, 0, 0)),
            pl.BlockSpec((1, 1, N), lambda b: (b, 0, 0)),
            pl.BlockSpec((1, M, 1), lambda b: (b, 0, 0)),
            pl.BlockSpec((1, M, 1), lambda b: (b, 0, 0)),
            pl.BlockSpec((1, M, 1), lambda b: (b, 0, 0)),
        ],
        out_specs=[
            pl.BlockSpec((1, M, N), lambda b: (b, 0, 0)),
            pl.BlockSpec((1, M, 16), lambda b: (b, 0, 0)),
        ],
        out_shape=[
            jax.ShapeDtypeStruct((nb, M, N), jnp.float32),
            jax.ShapeDtypeStruct((nb, M, 16), jnp.float32),
        ],
    )(x0, x1, x2, c0t, c1t, c2t)


_NC, _NS = 2, 16
_NW = _NC * _NS
_RPW = B * M // _NW


def _sc_filter_kernel(rpw, d_hbm, t_hbm, cd_hbm, ci_hbm,
                      d_v, t_v, cd_v, ci_v, semA, semB):
    wid = lax.axis_index("s") * _NC + lax.axis_index("c")
    r0 = wid * rpw
    inf16 = jnp.full((16,), _INF, jnp.float32)
    iota16 = lax.iota(jnp.int32, 16)

    pltpu.sync_copy(t_hbm.at[pl.ds(r0 * 16, rpw * 16)], t_v)

    def init_body(i, _):
        cd_v[pl.ds(i * 16, 16)] = inf16
        return 0

    lax.fori_loop(0, rpw * CAND // 16, init_body, 0)

    def filter_row(r, buf):
        tv = t_v[pl.ds(r * 16, 16)]
        obase = r * CAND
        nq = 8

        def chunk(j, cnt):
            base = j * (16 * nq)
            dvs = [d_v[buf, pl.ds(base + q * 16, 16)] for q in range(nq)]
            ms = [dv <= tv for dv in dvs]
            mors = ms[0]
            for q in range(1, nq):
                mors = mors | ms[q]
            npass = plsc.all_reduce_population_count(mors)
            anyv = npass[0] != 0

            def heavy(c):
                for q in range(nq):
                    mi = ms[q].astype(jnp.int32)
                    cs = plsc.cumsum(mi)
                    pos = c + cs - 1
                    ok = ms[q] & (pos < CAND)
                    plsc.store_scatter(cd_v, [obase + pos], dvs[q], mask=ok)
                    plsc.store_scatter(
                        ci_v, [obase + pos], base + q * 16 + iota16, mask=ok)
                    c = c + jnp.sum(mi)
                return c

            return lax.cond(anyv, heavy, lambda c: c, cnt)

        lax.fori_loop(0, N // (16 * nq), chunk, 0)
        return 0

    pltpu.async_copy(d_hbm.at[r0], d_v.at[0], semA)

    def pair(p, _):
        ra = r0 + 2 * p
        pltpu.async_copy(d_hbm.at[ra + 1], d_v.at[1], semB)
        pltpu.make_async_copy(d_hbm.at[ra], d_v.at[0], semA).wait()
        filter_row(2 * p, 0)
        rn = jnp.minimum(ra + 2, r0 + rpw - 1)
        pltpu.async_copy(d_hbm.at[rn], d_v.at[0], semA)
        pltpu.make_async_copy(
            d_hbm.at[ra + 1], d_v.at[1], semB).wait()
        filter_row(2 * p + 1, 1)
        return 0

    lax.fori_loop(0, rpw // 2, pair, 0)
    pltpu.make_async_copy(d_hbm.at[r0], d_v.at[0], semA).wait()

    pltpu.sync_copy(cd_v, cd_hbm.at[pl.ds(r0 * CAND, rpw * CAND)])
    pltpu.sync_copy(ci_v, ci_hbm.at[pl.ds(r0 * CAND, rpw * CAND)])


def _sc_filter(dflat, tflat):
    nrows = dflat.shape[0]
    rpw = nrows // _NW
    mesh = plsc.VectorSubcoreMesh(core_axis_name="c", subcore_axis_name="s")
    f = functools.partial(
        pl.kernel,
        mesh=mesh,
        compiler_params=pltpu.CompilerParams(needs_layout_passes=False),
        out_type=[
            jax.ShapeDtypeStruct((nrows * CAND,), jnp.float32),
            jax.ShapeDtypeStruct((nrows * CAND,), jnp.int32),
        ],
        scratch_types=[
            pltpu.VMEM((2, N), jnp.float32),
            pltpu.VMEM((rpw * 16,), jnp.float32),
            pltpu.VMEM((rpw * CAND,), jnp.float32),
            pltpu.VMEM((rpw * CAND,), jnp.int32),
            pltpu.SemaphoreType.DMA,
            pltpu.SemaphoreType.DMA,
        ],
    )(functools.partial(_sc_filter_kernel, rpw))
    return f(dflat, tflat)


def _sel_kernel(cd_ref, ci_ref, idx_ref):
    ic = ci_ref[0]
    colK = lax.broadcasted_iota(jnp.int32, (M, K), 1)

    def body(s, dc):
        minv = jnp.min(dc, axis=1, keepdims=True)
        sel = jnp.min(jnp.where(dc == minv, ic, N), axis=1, keepdims=True)
        dc = jnp.where((dc == minv) & (ic == sel), _INF, dc)
        idx_ref[0] = jnp.where(colK == s, sel, idx_ref[0])
        return dc

    lax.fori_loop(0, K, body, cd_ref[0])


def _sel(cd, ci):
    return pl.pallas_call(
        _sel_kernel,
        grid=(B,),
        in_specs=[
            pl.BlockSpec((1, M, CAND), lambda b: (b, 0, 0)),
            pl.BlockSpec((1, M, CAND), lambda b: (b, 0, 0)),
        ],
        out_specs=pl.BlockSpec((1, M, K), lambda b: (b, 0, 0)),
        out_shape=jax.ShapeDtypeStruct((B, M, K), jnp.int32),
    )(cd, ci)


_NC, _NS = 2, 16
_NW = _NC * _NS
_ROWS = B * M * K // _NW
_TPB = _NW // B
_D = 16


def _sc_gather_kernel(xpad_hbm, idx_hbm, out_hbm, idx_v, rows_v, sem):
    wid = lax.axis_index("s") * _NC + lax.axis_index("c")
    b = wid // _TPB
    base = b * (M * K) + (wid % _TPB) * _ROWS
    pltpu.sync_copy(idx_hbm.at[pl.ds(base, _ROWS)], idx_v)
    bn = b * N

    def body(j, _):
        idx_v[pl.ds(j * 16, 16)] = idx_v[pl.ds(j * 16, 16)] + bn
        return 0

    lax.fori_loop(0, _ROWS // 16, body, 0)
    pltpu.async_copy(xpad_hbm.at[idx_v], rows_v, sem).wait()
    pltpu.sync_copy(rows_v, out_hbm.at[pl.ds(base, _ROWS)])


def _sc_gather(xpad, idxf):
    mesh = plsc.VectorSubcoreMesh(core_axis_name="c", subcore_axis_name="s")
    f = functools.partial(
        pl.kernel,
        mesh=mesh,
        compiler_params=pltpu.CompilerParams(use_tc_tiling_on_sc=False),
        out_type=jax.ShapeDtypeStruct((B * M * K, _D), jnp.float32),
        scratch_types=[
            pltpu.VMEM((_ROWS,), jnp.int32),
            pltpu.VMEM((_ROWS, _D), jnp.float32),
            pltpu.SemaphoreType.DMA,
        ],
    )(_sc_gather_kernel)
    return f(xpad, idxf)


def kernel(x):
    x0 = x[:, :, 0]
    x1 = x[:, :, 1]
    x2 = x[:, :, 2]
    c0, c1, c2 = _fps(x0, x1, x2)
    cds, cis = [], []
    for h in range(4):
        sl = slice(h * (B // 4), (h + 1) * (B // 4))
        dm, tm = _dist(
            x0[sl, None, :], x1[sl, None, :], x2[sl, None, :],
            c0[sl, :, None], c1[sl, :, None], c2[sl, :, None],
        )
        cdh, cih = _sc_filter(dm.reshape(B // 4 * M, N), tm.reshape(-1))
        cds.append(cdh)
        cis.append(cih)
    cd = jnp.concatenate(cds)
    ci = jnp.concatenate(cis)
    idx = _sel(
        cd.reshape(B, M, CAND), ci.reshape(B, M, CAND)
    )
    xpad = jnp.pad(x.reshape(B * N, 3), ((0, 0), (0, _D - 3)))
    out = _sc_gather(xpad, idx.reshape(-1))
    return out[:, :3].reshape(B, M, K, 3)
